# baseline (speedup 1.0000x reference)
import functools

import jax
import jax.numpy as jnp
from jax import lax
from jax.experimental import pallas as pl
from jax.experimental.pallas import tpu as pltpu
from jax.experimental.pallas import tpu_sc as plsc

N = 10000
E = 320000
D = 128

NC = 2
NS = 16
NW = NC * NS

C = 128

CPW = 80
NCHUNKP = NW * CPW
EPAD = NCHUNKP * C

CPW2 = 160
NCHUNK2P = NW * CPW2
E2PAD = NCHUNK2P * C

NPAD = 10128
HPAD = 20480

_ROWS_PER_SUB = NPAD // NS
_HROWS_PER_SUB = HPAD // NS

_SC_PARAMS = pltpu.CompilerParams(use_tc_tiling_on_sc=False)


def _worker_id():
  return lax.axis_index("s") * NC + lax.axis_index("c")


_DBATCH = 8


def _deg_body(idx_hbm, zeros_hbm, ones_hbm, out_hbm, idx_v, ones_v, hist, sem):
  c = lax.axis_index("c")
  s = lax.axis_index("s")
  wid = _worker_id()

  pltpu.sync_copy(zeros_hbm.at[pl.ds(s * _HROWS_PER_SUB, _HROWS_PER_SUB)],
                  hist.at[pl.ds(s * _HROWS_PER_SUB, _HROWS_PER_SUB)])
  pltpu.sync_copy(ones_hbm, ones_v)
  pltpu.sync_copy(idx_hbm.at[pl.ds(wid * CPW2, CPW2)], idx_v)
  plsc.subcore_barrier()

  def body(g, carry):
    for b in range(_DBATCH):
      pltpu.async_copy(ones_v, hist.at[idx_v.at[g * _DBATCH + b]], sem,
                       add=True)
    for b in range(_DBATCH):
      pltpu.make_async_copy(ones_v, hist.at[idx_v.at[g * _DBATCH + b]],
                            sem).wait()
    return carry

  lax.fori_loop(0, CPW2 // _DBATCH, body, 0, unroll=False)
  plsc.subcore_barrier()

  pltpu.sync_copy(hist.at[pl.ds(s * _HROWS_PER_SUB, _HROWS_PER_SUB)],
                  out_hbm.at[c, pl.ds(s * _HROWS_PER_SUB, _HROWS_PER_SUB)])


_deg_kernel = functools.partial(
    pl.kernel,
    out_type=jax.ShapeDtypeStruct((NC, HPAD, 16), jnp.float32),
    mesh=plsc.VectorSubcoreMesh(core_axis_name="c", subcore_axis_name="s"),
    scratch_types=[
        pltpu.VMEM((CPW2, C), jnp.int32),
        pltpu.VMEM((C, 16), jnp.float32),
        pltpu.VMEM_SHARED((HPAD, 16), jnp.float32),
        pltpu.SemaphoreType.DMA,
    ],
    compiler_params=_SC_PARAMS,
)(_deg_body)


CA = 64
CPWA = EPAD // (NW * CA)
QC = CPWA // 8
NB = 5


def _agg_body(h_hbm, src_hbm, dst_hbm, zeros_hbm, out_hbm,
              src_v, dst_v, rows, sems, hist):
  c = lax.axis_index("c")
  s = lax.axis_index("s")
  wid = _worker_id()

  pltpu.sync_copy(zeros_hbm.at[pl.ds(s * _ROWS_PER_SUB, _ROWS_PER_SUB)],
                  hist.at[pl.ds(s * _ROWS_PER_SUB, _ROWS_PER_SUB)])
  plsc.subcore_barrier()

  def slab(q, carry):
    base = wid * CPWA + q * QC
    pltpu.sync_copy(src_hbm.at[pl.ds(base, QC)], src_v)
    pltpu.sync_copy(dst_hbm.at[pl.ds(base, QC)], dst_v)
    for b in range(NB - 1):
      pltpu.async_copy(h_hbm.at[src_v.at[b]], rows[b], sems[b])

    def body(g, carry2):
      for b in range(NB):
        j = g * NB + b
        pltpu.make_async_copy(h_hbm.at[src_v.at[j]], rows[b], sems[b]).wait()

        @pl.when(j + NB - 1 < QC)
        def _():
          pltpu.async_copy(h_hbm.at[src_v.at[j + NB - 1]],
                           rows[(b + NB - 1) % NB], sems[(b + NB - 1) % NB])

        pltpu.sync_copy(rows[b], hist.at[dst_v.at[j]], add=True)
      return carry2

    lax.fori_loop(0, QC // NB, body, 0, unroll=False)
    return carry

  lax.fori_loop(0, 8, slab, 0, unroll=False)
  plsc.subcore_barrier()

  pltpu.sync_copy(hist.at[pl.ds(s * _ROWS_PER_SUB, _ROWS_PER_SUB)],
                  out_hbm.at[c, pl.ds(s * _ROWS_PER_SUB, _ROWS_PER_SUB)])


_agg_kernel = functools.partial(
    pl.kernel,
    out_type=jax.ShapeDtypeStruct((NC, NPAD, D), jnp.float32),
    mesh=plsc.VectorSubcoreMesh(core_axis_name="c", subcore_axis_name="s"),
    scratch_types=[
        pltpu.VMEM((QC, CA), jnp.int32),
        pltpu.VMEM((QC, CA), jnp.int32),
        [pltpu.VMEM((CA, D), jnp.float32)] * NB,
        [pltpu.SemaphoreType.DMA] * NB,
        pltpu.VMEM_SHARED((NPAD, D), jnp.float32),
    ],
    compiler_params=_SC_PARAMS,
)(_agg_body)


_RB = 2000


def _norm_scale_body(do0, do1, di0, di1, x, h_out, nd_out):
  deg_out = do0[0, :, 0] + do1[0, :, 0]
  deg_in = di0[0, :, 0] + di1[0, :, 0]
  norm_src = lax.rsqrt(jnp.maximum(deg_out, 1.0))
  norm_dst = lax.rsqrt(jnp.maximum(deg_in, 1.0))
  h_out[...] = x[...] * norm_src[:, None]
  nd_out[...] = norm_dst[:, None]


def _norm_scale(hist_parts, in_feat):
  nb = N // _RB
  return pl.pallas_call(
      _norm_scale_body,
      grid=(nb,),
      in_specs=[
          pl.BlockSpec((1, _RB, 16), lambda i: (0, i, 0)),
          pl.BlockSpec((1, _RB, 16), lambda i: (1, i, 0)),
          pl.BlockSpec((1, _RB, 16), lambda i: (0, i + nb, 0)),
          pl.BlockSpec((1, _RB, 16), lambda i: (1, i + nb, 0)),
          pl.BlockSpec((_RB, D), lambda i: (i, 0)),
      ],
      out_specs=[
          pl.BlockSpec((_RB, D), lambda i: (i, 0)),
          pl.BlockSpec((_RB, 1), lambda i: (i, 0)),
      ],
      out_shape=[
          jax.ShapeDtypeStruct((N, D), jnp.float32),
          jax.ShapeDtypeStruct((N, 1), jnp.float32),
      ],
  )(hist_parts, hist_parts, hist_parts, hist_parts, in_feat)


def _out_body(p0, p1, nd, w, bias, out):
  a = (p0[0] + p1[0]) * nd[...]
  y = jnp.dot(a, w[...], preferred_element_type=jnp.float32) + bias[...]
  out[...] = jnp.maximum(y, 0.0)


def _final(agg_parts, norm_dst, W, b2d):
  nb = N // _RB
  return pl.pallas_call(
      _out_body,
      grid=(nb,),
      in_specs=[
          pl.BlockSpec((1, _RB, D), lambda i: (0, i, 0)),
          pl.BlockSpec((1, _RB, D), lambda i: (1, i, 0)),
          pl.BlockSpec((_RB, 1), lambda i: (i, 0)),
          pl.BlockSpec((D, D), lambda i: (0, 0)),
          pl.BlockSpec((1, D), lambda i: (0, 0)),
      ],
      out_specs=pl.BlockSpec((_RB, D), lambda i: (i, 0)),
      out_shape=jax.ShapeDtypeStruct((N, D), jnp.float32),
  )(agg_parts, agg_parts, norm_dst, W, b2d)


@jax.jit
def kernel(in_feat, edge_index, W, b):
  src = edge_index[0]
  dst = edge_index[1]
  cyc = jnp.arange(EPAD - E, dtype=jnp.int32) % 128
  cyc2 = jnp.arange(E2PAD - 2 * E, dtype=jnp.int32) % 128
  src_pad = jnp.concatenate([src, cyc]).reshape(EPAD // CA, CA)
  dst_pad = jnp.concatenate([dst, N + cyc]).reshape(EPAD // CA, CA)
  idx_all = jnp.concatenate([src, dst + N, 2 * N + cyc2]).reshape(NCHUNK2P, C)

  zeros_hist = jnp.zeros((HPAD, 16), jnp.float32)
  ones_c = jnp.zeros((C, 16), jnp.float32).at[:, 0].set(1.0)
  zeros_agg = jnp.zeros((NPAD, D), jnp.float32)

  hist_parts = _deg_kernel(idx_all, zeros_hist, ones_c)
  h, norm_dst = _norm_scale(hist_parts, in_feat)
  agg_parts = _agg_kernel(h, src_pad, dst_pad, zeros_agg)
  return _final(agg_parts, norm_dst, W, b.reshape(1, D))


# speedup vs baseline: 1.0925x; 1.0925x over previous
import functools

import jax
import jax.numpy as jnp
from jax import lax
from jax.experimental import pallas as pl
from jax.experimental.pallas import tpu as pltpu
from jax.experimental.pallas import tpu_sc as plsc

N = 10000
E = 320000
D = 128

NC = 2
NS = 16
NW = NC * NS

C = 128

CPW = 80
NCHUNKP = NW * CPW
EPAD = NCHUNKP * C

CPW2 = 160
NCHUNK2P = NW * CPW2
E2PAD = NCHUNK2P * C

NPAD = 10128
HPAD = 20480

_ROWS_PER_SUB = NPAD // NS
_HROWS_PER_SUB = HPAD // NS

_SC_PARAMS = pltpu.CompilerParams(use_tc_tiling_on_sc=False)


def _worker_id():
  return lax.axis_index("s") * NC + lax.axis_index("c")


_DBATCH = 8


def _deg_body(idx_hbm, zeros_hbm, ones_hbm, out_hbm, idx_v, ones_v, hist, sem):
  c = lax.axis_index("c")
  s = lax.axis_index("s")
  wid = _worker_id()

  pltpu.sync_copy(zeros_hbm.at[pl.ds(s * _HROWS_PER_SUB, _HROWS_PER_SUB)],
                  hist.at[pl.ds(s * _HROWS_PER_SUB, _HROWS_PER_SUB)])
  pltpu.sync_copy(ones_hbm, ones_v)
  pltpu.sync_copy(idx_hbm.at[pl.ds(wid * CPW2, CPW2)], idx_v)
  plsc.subcore_barrier()

  def body(g, carry):
    for b in range(_DBATCH):
      pltpu.async_copy(ones_v, hist.at[idx_v.at[g * _DBATCH + b]], sem,
                       add=True)
    for b in range(_DBATCH):
      pltpu.make_async_copy(ones_v, hist.at[idx_v.at[g * _DBATCH + b]],
                            sem).wait()
    return carry

  lax.fori_loop(0, CPW2 // _DBATCH, body, 0, unroll=False)
  plsc.subcore_barrier()

  pltpu.sync_copy(hist.at[pl.ds(s * _HROWS_PER_SUB, _HROWS_PER_SUB)],
                  out_hbm.at[c, pl.ds(s * _HROWS_PER_SUB, _HROWS_PER_SUB)])


_deg_kernel = functools.partial(
    pl.kernel,
    out_type=jax.ShapeDtypeStruct((NC, HPAD, 16), jnp.float32),
    mesh=plsc.VectorSubcoreMesh(core_axis_name="c", subcore_axis_name="s"),
    scratch_types=[
        pltpu.VMEM((CPW2, C), jnp.int32),
        pltpu.VMEM((C, 16), jnp.float32),
        pltpu.VMEM_SHARED((HPAD, 16), jnp.float32),
        pltpu.SemaphoreType.DMA,
    ],
    compiler_params=_SC_PARAMS,
)(_deg_body)


CA = 64
CPWA = EPAD // (NW * CA)
QC = CPWA // 2
NB = 4


def _agg_body(h_hbm, src_hbm, dst_hbm, zeros_hbm, out_hbm,
              src_v, dst_v, rows, sems, hist):
  c = lax.axis_index("c")
  s = lax.axis_index("s")
  wid = _worker_id()

  pltpu.sync_copy(zeros_hbm.at[pl.ds(s * _ROWS_PER_SUB, _ROWS_PER_SUB)],
                  hist.at[pl.ds(s * _ROWS_PER_SUB, _ROWS_PER_SUB)])
  plsc.subcore_barrier()

  def quarter(q, carry):
    base = wid * CPWA + q * QC
    pltpu.sync_copy(src_hbm.at[pl.ds(base, QC)], src_v)
    pltpu.sync_copy(dst_hbm.at[pl.ds(base, QC)], dst_v)
    for b in range(NB - 1):
      pltpu.async_copy(h_hbm.at[src_v.at[b]], rows[b], sems[b])

    def body(g, carry2):
      for b in range(NB):
        j = g * NB + b
        pltpu.make_async_copy(h_hbm.at[src_v.at[j]], rows[b], sems[b]).wait()

        @pl.when(j + NB - 1 < QC)
        def _():
          pltpu.async_copy(h_hbm.at[src_v.at[j + NB - 1]],
                           rows[(b + NB - 1) % NB], sems[(b + NB - 1) % NB])

        pltpu.sync_copy(rows[b], hist.at[dst_v.at[j]], add=True)
      return carry2

    lax.fori_loop(0, QC // NB, body, 0, unroll=False)
    return carry

  lax.fori_loop(0, 2, quarter, 0, unroll=False)
  plsc.subcore_barrier()

  pltpu.sync_copy(hist.at[pl.ds(s * _ROWS_PER_SUB, _ROWS_PER_SUB)],
                  out_hbm.at[c, pl.ds(s * _ROWS_PER_SUB, _ROWS_PER_SUB)])


_agg_kernel = functools.partial(
    pl.kernel,
    out_type=jax.ShapeDtypeStruct((NC, NPAD, D), jnp.float32),
    mesh=plsc.VectorSubcoreMesh(core_axis_name="c", subcore_axis_name="s"),
    scratch_types=[
        pltpu.VMEM((QC, CA), jnp.int32),
        pltpu.VMEM((QC, CA), jnp.int32),
        [pltpu.VMEM((CA, D), jnp.float32)] * NB,
        [pltpu.SemaphoreType.DMA] * NB,
        pltpu.VMEM_SHARED((NPAD, D), jnp.float32),
    ],
    compiler_params=_SC_PARAMS,
)(_agg_body)


_RB = 2000


def _norm_scale_body(do0, do1, di0, di1, x, h_out, nd_out):
  deg_out = do0[0, :, 0] + do1[0, :, 0]
  deg_in = di0[0, :, 0] + di1[0, :, 0]
  norm_src = lax.rsqrt(jnp.maximum(deg_out, 1.0))
  norm_dst = lax.rsqrt(jnp.maximum(deg_in, 1.0))
  h_out[...] = x[...] * norm_src[:, None]
  nd_out[...] = norm_dst[:, None]


def _norm_scale(hist_parts, in_feat):
  nb = N // _RB
  return pl.pallas_call(
      _norm_scale_body,
      grid=(nb,),
      in_specs=[
          pl.BlockSpec((1, _RB, 16), lambda i: (0, i, 0)),
          pl.BlockSpec((1, _RB, 16), lambda i: (1, i, 0)),
          pl.BlockSpec((1, _RB, 16), lambda i: (0, i + nb, 0)),
          pl.BlockSpec((1, _RB, 16), lambda i: (1, i + nb, 0)),
          pl.BlockSpec((_RB, D), lambda i: (i, 0)),
      ],
      out_specs=[
          pl.BlockSpec((_RB, D), lambda i: (i, 0)),
          pl.BlockSpec((_RB, 1), lambda i: (i, 0)),
      ],
      out_shape=[
          jax.ShapeDtypeStruct((N, D), jnp.float32),
          jax.ShapeDtypeStruct((N, 1), jnp.float32),
      ],
  )(hist_parts, hist_parts, hist_parts, hist_parts, in_feat)


def _out_body(p0, p1, nd, w, bias, out):
  a = (p0[0] + p1[0]) * nd[...]
  y = jnp.dot(a, w[...], preferred_element_type=jnp.float32) + bias[...]
  out[...] = jnp.maximum(y, 0.0)


def _final(agg_parts, norm_dst, W, b2d):
  nb = N // _RB
  return pl.pallas_call(
      _out_body,
      grid=(nb,),
      in_specs=[
          pl.BlockSpec((1, _RB, D), lambda i: (0, i, 0)),
          pl.BlockSpec((1, _RB, D), lambda i: (1, i, 0)),
          pl.BlockSpec((_RB, 1), lambda i: (i, 0)),
          pl.BlockSpec((D, D), lambda i: (0, 0)),
          pl.BlockSpec((1, D), lambda i: (0, 0)),
      ],
      out_specs=pl.BlockSpec((_RB, D), lambda i: (i, 0)),
      out_shape=jax.ShapeDtypeStruct((N, D), jnp.float32),
  )(agg_parts, agg_parts, norm_dst, W, b2d)


@jax.jit
def kernel(in_feat, edge_index, W, b):
  src = edge_index[0]
  dst = edge_index[1]
  cyc = jnp.arange(EPAD - E, dtype=jnp.int32) % 128
  cyc2 = jnp.arange(E2PAD - 2 * E, dtype=jnp.int32) % 128
  src_pad = jnp.concatenate([src, cyc]).reshape(EPAD // CA, CA)
  dst_pad = jnp.concatenate([dst, N + cyc]).reshape(EPAD // CA, CA)
  idx_all = jnp.concatenate([src, dst + N, 2 * N + cyc2]).reshape(NCHUNK2P, C)

  zeros_hist = jnp.zeros((HPAD, 16), jnp.float32)
  ones_c = jnp.zeros((C, 16), jnp.float32).at[:, 0].set(1.0)
  zeros_agg = jnp.zeros((NPAD, D), jnp.float32)

  hist_parts = _deg_kernel(idx_all, zeros_hist, ones_c)
  h, norm_dst = _norm_scale(hist_parts, in_feat)
  agg_parts = _agg_kernel(h, src_pad, dst_pad, zeros_agg)
  return _final(agg_parts, norm_dst, W, b.reshape(1, D))
